# baseline (speedup 1.0000x reference)
import jax
import jax.numpy as jnp
from jax.experimental import pallas as pl
from jax.experimental.pallas import tpu as pltpu

_BN = 5000


def _phase1_kernel(adj_ref, emb_ref, w_ref, g1_ref, b1_ref, lat1_ref, acc_ref):
    i = pl.program_id(0)

    @pl.when(i == 0)
    def _init():
        acc_ref[...] = jnp.zeros_like(acc_ref)

    a = adj_ref[...].astype(jnp.bfloat16)
    e = emb_ref[...].astype(jnp.bfloat16)
    acc_ref[...] += jax.lax.dot_general(
        a, e, (((0,), (0,)), ((), ())), preferred_element_type=jnp.float32
    )

    @pl.when(i == pl.num_programs(0) - 1)
    def _finish():
        s = acc_ref[...].astype(jnp.bfloat16)
        x = jax.lax.dot_general(
            s, w_ref[...], (((1,), (0,)), ((), ())),
            preferred_element_type=jnp.float32,
        )
        m = jnp.mean(x, axis=-1, keepdims=True)
        v = jnp.mean((x - m) ** 2, axis=-1, keepdims=True)
        y = (x - m) * jax.lax.rsqrt(v + 1e-5) * g1_ref[...] + b1_ref[...]
        lat1_ref[...] = y.astype(jnp.bfloat16)


def _phase2_kernel(adj_ref, lat1_ref, g2_ref, b2_ref, out_ref):
    a = adj_ref[...].astype(jnp.bfloat16)
    y = jax.lax.dot_general(
        a, lat1_ref[...], (((1,), (0,)), ((), ())),
        preferred_element_type=jnp.float32,
    )
    m = jnp.mean(y, axis=-1, keepdims=True)
    v = jnp.mean((y - m) ** 2, axis=-1, keepdims=True)
    z = (y - m) * jax.lax.rsqrt(v + 1e-5) * g2_ref[...] + b2_ref[...]
    out_ref[...] = jnp.where(z >= 0, z, 0.2 * z)


def kernel(adj, embeds, W, g1, b1, g2, b2):
    n, h = adj.shape
    d = embeds.shape[1]
    dh = W.shape[1]
    bn = _BN if n % _BN == 0 else n
    num_blocks = n // bn

    w_bf = W.astype(jnp.bfloat16)
    g1r, b1r = g1.reshape(1, dh), b1.reshape(1, dh)
    g2r, b2r = g2.reshape(1, dh), b2.reshape(1, dh)

    lat1 = pl.pallas_call(
        _phase1_kernel,
        grid=(num_blocks,),
        in_specs=[
            pl.BlockSpec((bn, h), lambda i: (i, 0)),
            pl.BlockSpec((bn, d), lambda i: (i, 0)),
            pl.BlockSpec((d, dh), lambda i: (0, 0)),
            pl.BlockSpec((1, dh), lambda i: (0, 0)),
            pl.BlockSpec((1, dh), lambda i: (0, 0)),
        ],
        out_specs=pl.BlockSpec((h, dh), lambda i: (0, 0)),
        out_shape=jax.ShapeDtypeStruct((h, dh), jnp.bfloat16),
        scratch_shapes=[pltpu.VMEM((h, d), jnp.float32)],
        compiler_params=pltpu.CompilerParams(
            dimension_semantics=("arbitrary",),
        ),
    )(adj, embeds, w_bf, g1r, b1r)

    out = pl.pallas_call(
        _phase2_kernel,
        grid=(num_blocks,),
        in_specs=[
            pl.BlockSpec((bn, h), lambda i: (i, 0)),
            pl.BlockSpec((h, dh), lambda i: (0, 0)),
            pl.BlockSpec((1, dh), lambda i: (0, 0)),
            pl.BlockSpec((1, dh), lambda i: (0, 0)),
        ],
        out_specs=pl.BlockSpec((bn, dh), lambda i: (i, 0)),
        out_shape=jax.ShapeDtypeStruct((n, dh), jnp.float32),
        compiler_params=pltpu.CompilerParams(
            dimension_semantics=("arbitrary",),
        ),
    )(adj, lat1, g2r, b2r)

    return out


# speedup vs baseline: 1.0357x; 1.0357x over previous
import jax
import jax.numpy as jnp
from jax.experimental import pallas as pl
from jax.experimental.pallas import tpu as pltpu

_BN = 2000


def _fused_kernel(adj_ref, emb_ref, w_ref, g1_ref, b1_ref, g2_ref, b2_ref,
                  out_ref, acc_ref, lat1_ref):
    i = pl.program_id(0)
    nb = pl.num_programs(0) // 2

    @pl.when(i == 0)
    def _init():
        acc_ref[...] = jnp.zeros_like(acc_ref)

    @pl.when(i < nb)
    def _phase1():
        a = adj_ref[...].astype(jnp.bfloat16)
        e = emb_ref[...].astype(jnp.bfloat16)
        acc_ref[...] += jax.lax.dot_general(
            a, e, (((0,), (0,)), ((), ())), preferred_element_type=jnp.float32
        )

    @pl.when(i == nb - 1)
    def _mid():
        s = acc_ref[...].astype(jnp.bfloat16)
        x = jax.lax.dot_general(
            s, w_ref[...], (((1,), (0,)), ((), ())),
            preferred_element_type=jnp.float32,
        )
        m = jnp.mean(x, axis=-1, keepdims=True)
        v = jnp.mean((x - m) ** 2, axis=-1, keepdims=True)
        y = (x - m) * jax.lax.rsqrt(v + 1e-5) * g1_ref[...] + b1_ref[...]
        lat1_ref[...] = y.astype(jnp.bfloat16)

    @pl.when(i >= nb)
    def _phase2():
        a = adj_ref[...].astype(jnp.bfloat16)
        y = jax.lax.dot_general(
            a, lat1_ref[...], (((1,), (0,)), ((), ())),
            preferred_element_type=jnp.float32,
        )
        m = jnp.mean(y, axis=-1, keepdims=True)
        v = jnp.mean((y - m) ** 2, axis=-1, keepdims=True)
        z = (y - m) * jax.lax.rsqrt(v + 1e-5) * g2_ref[...] + b2_ref[...]
        out_ref[...] = jnp.where(z >= 0, z, 0.2 * z)


def kernel(adj, embeds, W, g1, b1, g2, b2):
    n, h = adj.shape
    d = embeds.shape[1]
    dh = W.shape[1]
    bn = _BN if n % _BN == 0 else n
    nb = n // bn

    w_bf = W.astype(jnp.bfloat16)
    g1r, b1r = g1.reshape(1, dh), b1.reshape(1, dh)
    g2r, b2r = g2.reshape(1, dh), b2.reshape(1, dh)

    out = pl.pallas_call(
        _fused_kernel,
        grid=(2 * nb,),
        in_specs=[
            pl.BlockSpec((bn, h), lambda i: (jnp.where(i < nb, i, 2 * nb - 1 - i), 0)),
            pl.BlockSpec((bn, d), lambda i: (jnp.where(i < nb, i, 0), 0)),
            pl.BlockSpec((d, dh), lambda i: (0, 0)),
            pl.BlockSpec((1, dh), lambda i: (0, 0)),
            pl.BlockSpec((1, dh), lambda i: (0, 0)),
            pl.BlockSpec((1, dh), lambda i: (0, 0)),
            pl.BlockSpec((1, dh), lambda i: (0, 0)),
        ],
        out_specs=pl.BlockSpec(
            (bn, dh), lambda i: (jnp.where(i < nb, nb - 1, 2 * nb - 1 - i), 0)
        ),
        out_shape=jax.ShapeDtypeStruct((n, dh), jnp.float32),
        scratch_shapes=[
            pltpu.VMEM((h, d), jnp.float32),
            pltpu.VMEM((h, dh), jnp.bfloat16),
        ],
        compiler_params=pltpu.CompilerParams(
            dimension_semantics=("arbitrary",),
        ),
    )(adj, embeds, w_bf, g1r, b1r, g2r, b2r)

    return out
